# Initial kernel scaffold; baseline (speedup 1.0000x reference)
import jax
import jax.numpy as jnp
from jax.experimental import pallas as pl


def kernel(x, logits):
    raise NotImplementedError("write your pallas kernel here")


# speedup vs baseline: 308.8455x; 308.8455x over previous
import functools

import jax
import jax.numpy as jnp
from jax import lax
from jax.experimental import pallas as pl
from jax.experimental.pallas import tpu as pltpu
from jax.experimental.pallas import tpu_sc as plsc

_N_NODES = 100
_N_STATES = 1000
_BATCH = 16384

_NW = 32
_SPW = _BATCH // _NW
_CH = 128
_NCHUNK = _SPW // _CH
_GRP = _CH // 16


def _lse_body(logits_ref, out_ref):
    l = logits_ref[...]
    m = jnp.max(l, axis=1, keepdims=True)
    s = jnp.sum(jnp.exp(l - m), axis=1, keepdims=True)
    lse = jnp.log(s) + m
    out_ref[...] = jnp.full((8, 128), jnp.sum(lse), jnp.float32)


def _lse_total(logits):
    return pl.pallas_call(
        _lse_body,
        out_shape=jax.ShapeDtypeStruct((8, 128), jnp.float32),
    )(logits)


def _sc_gather_sum(x_flat, tab_flat, lse_flat):
    mesh = plsc.VectorSubcoreMesh(core_axis_name="c", subcore_axis_name="s")

    @functools.partial(
        pl.kernel,
        mesh=mesh,
        out_type=jax.ShapeDtypeStruct((_BATCH,), jnp.float32),
        compiler_params=pltpu.CompilerParams(needs_layout_passes=False),
        scratch_types=[
            pltpu.VMEM((_N_NODES * _N_STATES,), jnp.float32),
            pltpu.VMEM((_CH * _N_NODES,), jnp.int32),
            pltpu.VMEM((_CH * _N_NODES,), jnp.int32),
            pltpu.VMEM((_CH,), jnp.float32),
            pltpu.VMEM((16,), jnp.float32),
            pltpu.SemaphoreType.DMA,
            pltpu.SemaphoreType.DMA,
            pltpu.SemaphoreType.DMA,
            pltpu.SemaphoreType.DMA,
            pltpu.SemaphoreType.DMA,
        ],
    )
    def k(x_hbm, tab_hbm, lse_hbm, out_hbm,
          tab_v, xa_v, xb_v, out_v, lse_v,
          sem_t, sem_x0, sem_x1, sem_l, sem_o):
        wid = lax.axis_index("s") * 2 + lax.axis_index("c")
        base = wid * _SPW

        h_t = pltpu.async_copy(tab_hbm, tab_v, sem_t)
        h_l = pltpu.async_copy(lse_hbm.at[pl.ds(0, 16)], lse_v, sem_l)
        xbufs = (xa_v, xb_v)
        xsems = (sem_x0, sem_x1)
        h = [None, None]
        h[0] = pltpu.async_copy(
            x_hbm.at[pl.ds(base * _N_NODES, _CH * _N_NODES)], xa_v, sem_x0)
        h_t.wait()
        h_l.wait()
        inv_n = jnp.float32(1.0 / _N_NODES)
        lse_s = lse_v[...] * inv_n
        iota = lax.iota(jnp.int32, 16)

        for c in range(_NCHUNK):
            if c + 1 < _NCHUNK:
                h[(c + 1) % 2] = pltpu.async_copy(
                    x_hbm.at[pl.ds((base + (c + 1) * _CH) * _N_NODES,
                                   _CH * _N_NODES)],
                    xbufs[(c + 1) % 2], xsems[(c + 1) % 2])
            h[c % 2].wait()
            xv = xbufs[c % 2]
            for g in range(_GRP):
                bvec = iota * _N_NODES + g * (16 * _N_NODES)

                def body(n, acc, bvec=bvec, xv=xv):
                    xcol = plsc.load_gather(xv, [bvec + n])
                    val = plsc.load_gather(tab_v, [xcol + n * _N_STATES])
                    return acc + val

                acc = lax.fori_loop(0, _N_NODES, body,
                                    jnp.zeros((16,), jnp.float32), unroll=4)
                out_v[pl.ds(g * 16, 16)] = acc * inv_n - lse_s
            pltpu.async_copy(
                out_v, out_hbm.at[pl.ds(base + c * _CH, _CH)], sem_o).wait()

    return k(x_flat, tab_flat, lse_flat)


def kernel(x, logits):
    lse_flat = _lse_total(logits).reshape(-1)
    return _sc_gather_sum(x.reshape(-1), logits.reshape(-1), lse_flat)
